# baseline (speedup 1.0000x reference)
import functools

import jax
import jax.numpy as jnp
from jax import lax
from jax.experimental import pallas as pl
from jax.experimental.pallas import tpu as pltpu
from jax.experimental.pallas import tpu_sc as plsc

_B = 16384
_V = 1000
_D = 128
_NC = 2
_NS = 16
_NW = _NC * _NS
_BPW = _B // _NW

_CHUNKS = (16, 48, 64, 64, 64, 64, 64, 64, 48, 16)
_OFFS = tuple(sum(_CHUNKS[:i]) for i in range(len(_CHUNKS)))
_HBM_CHUNKS = (0, 1, 9)
_SH_CHUNKS = (2, 3, 4, 5, 6, 7, 8)

_mesh = plsc.VectorSubcoreMesh(core_axis_name="c", subcore_axis_name="s")


@functools.partial(
    pl.kernel,
    mesh=_mesh,
    out_type=jax.ShapeDtypeStruct((_B, _D), jnp.float32),
    scratch_types=[
        pltpu.VMEM((_BPW,), jnp.int32),
        pltpu.VMEM((_BPW, _D), jnp.float32),
        pltpu.VMEM_SHARED((_V, _D), jnp.float32),
        pltpu.SemaphoreType.DMA((len(_CHUNKS),)),
        pltpu.SemaphoreType.DMA,
    ],
)
def _gather_kernel(t_hbm, table_hbm, out_hbm, idx_v, rows_v, table_sh, gsems, ssem):
    cid = lax.axis_index("c")
    sid = lax.axis_index("s")
    wid = sid * _NC + cid
    base = pl.multiple_of(wid * _BPW, _BPW)
    pltpu.sync_copy(t_hbm.at[pl.ds(base, _BPW)], idx_v)
    @pl.when(sid < 15)
    def _():
        off = pl.multiple_of(sid * 64, 8)
        pltpu.sync_copy(table_hbm.at[pl.ds(off, 64)], table_sh.at[pl.ds(off, 64)])

    @pl.when(sid == 15)
    def _():
        pltpu.sync_copy(table_hbm.at[pl.ds(960, 40)], table_sh.at[pl.ds(960, 40)])

    gathers = {}
    for j in _HBM_CHUNKS:
        gathers[j] = pltpu.async_copy(
            table_hbm.at[idx_v.at[pl.ds(_OFFS[j], _CHUNKS[j])]],
            rows_v.at[pl.ds(_OFFS[j], _CHUNKS[j])],
            gsems.at[j],
        )
    plsc.subcore_barrier()
    for j in _SH_CHUNKS:
        gathers[j] = pltpu.async_copy(
            table_sh.at[idx_v.at[pl.ds(_OFFS[j], _CHUNKS[j])]],
            rows_v.at[pl.ds(_OFFS[j], _CHUNKS[j])],
            gsems.at[j],
        )
    stores = []
    for j in _HBM_CHUNKS + _SH_CHUNKS:
        gathers[j].wait()
        stores.append(
            pltpu.async_copy(
                rows_v.at[pl.ds(_OFFS[j], _CHUNKS[j])],
                out_hbm.at[pl.ds(base + _OFFS[j], _CHUNKS[j])],
                ssem,
            )
        )
    for s in stores:
        s.wait()


def kernel(t, embed_weight):
    return _gather_kernel(t.astype(jnp.int32), embed_weight)


# speedup vs baseline: 1.0353x; 1.0353x over previous
import functools

import jax
import jax.numpy as jnp
from jax import lax
from jax.experimental import pallas as pl
from jax.experimental.pallas import tpu as pltpu
from jax.experimental.pallas import tpu_sc as plsc

_B = 16384
_V = 1000
_D = 128
_NC = 2
_NS = 16
_NW = _NC * _NS
_BPW = _B // _NW

_CHUNKS = (16, 48, 64, 64, 64, 64, 64, 64, 48, 16)
_OFFS = tuple(sum(_CHUNKS[:i]) for i in range(len(_CHUNKS)))

_mesh = plsc.VectorSubcoreMesh(core_axis_name="c", subcore_axis_name="s")


@functools.partial(
    pl.kernel,
    mesh=_mesh,
    out_type=jax.ShapeDtypeStruct((_B, _D), jnp.float32),
    scratch_types=[
        pltpu.VMEM((_BPW,), jnp.int32),
        pltpu.VMEM((_BPW, _D), jnp.float32),
        pltpu.VMEM_SHARED((_V, _D), jnp.float32),
        pltpu.SemaphoreType.DMA((len(_CHUNKS),)),
        pltpu.SemaphoreType.DMA,
    ],
)
def _gather_kernel(t_hbm, table_hbm, out_hbm, idx_v, rows_v, table_sh, gsems, ssem):
    cid = lax.axis_index("c")
    sid = lax.axis_index("s")
    wid = sid * _NC + cid
    base = pl.multiple_of(wid * _BPW, _BPW)
    idx_copy = pltpu.async_copy(t_hbm.at[pl.ds(base, _BPW)], idx_v, ssem)
    @pl.when(sid < 15)
    def _():
        off = pl.multiple_of(sid * 64, 8)
        pltpu.sync_copy(table_hbm.at[pl.ds(off, 64)], table_sh.at[pl.ds(off, 64)])

    @pl.when(sid == 15)
    def _():
        pltpu.sync_copy(table_hbm.at[pl.ds(960, 40)], table_sh.at[pl.ds(960, 40)])

    idx_copy.wait()
    plsc.subcore_barrier()
    gathers = [
        pltpu.async_copy(
            table_sh.at[idx_v.at[pl.ds(_OFFS[j], _CHUNKS[j])]],
            rows_v.at[pl.ds(_OFFS[j], _CHUNKS[j])],
            gsems.at[j],
        )
        for j in range(len(_CHUNKS))
    ]
    stores = []
    for j in range(len(_CHUNKS)):
        gathers[j].wait()
        stores.append(
            pltpu.async_copy(
                rows_v.at[pl.ds(_OFFS[j], _CHUNKS[j])],
                out_hbm.at[pl.ds(base + _OFFS[j], _CHUNKS[j])],
                ssem,
            )
        )
    for s in stores:
        s.wait()


def kernel(t, embed_weight):
    return _gather_kernel(t.astype(jnp.int32), embed_weight)
